# baseline (speedup 1.0000x reference)
import functools

import jax
import jax.numpy as jnp
from jax import lax
from jax.experimental import pallas as pl
from jax.experimental.pallas import tpu as pltpu

BR = 512


def _tobf16_body(A_ref, Ab_ref):
    Ab_ref[...] = A_ref[...].astype(jnp.bfloat16)


def _tobf16(A):
    N = A.shape[0]
    return pl.pallas_call(
        _tobf16_body,
        grid=(N // BR,),
        in_specs=[pl.BlockSpec((BR, N), lambda i: (i, 0))],
        out_specs=pl.BlockSpec((BR, N), lambda i: (i, 0)),
        out_shape=jax.ShapeDtypeStruct((N, N), jnp.bfloat16),
    )(A)


def _gat_body(Z_ref, A_ref, W_ref, as_ref, ar_ref, out_ref, C_ref,
              Hb_ref, f1_ref, f2_ref, Eb_ref):
    i = pl.program_id(0)
    dout = out_ref.shape[1]

    @pl.when(i == 0)
    def _prologue():
        H = jnp.dot(Z_ref[...], W_ref[...], preferred_element_type=jnp.float32)
        n = H.shape[0]
        ones_col = (lax.broadcasted_iota(jnp.int32, (n, 128), 1) == 0)
        Hb_ref[...] = jnp.concatenate(
            [H.astype(jnp.bfloat16), ones_col.astype(jnp.bfloat16)], axis=1)
        f1_ref[...] = 0.5 * lax.dot_general(
            H, as_ref[...], (((1,), (1,)), ((), ())),
            preferred_element_type=jnp.float32)
        f2_ref[...] = 0.5 * lax.dot_general(
            ar_ref[...], H, (((1,), (1,)), ((), ())),
            preferred_element_type=jnp.float32)

    f1b = f1_ref[pl.ds(i * BR, BR), :]
    e = 0.5 * jnp.tanh(f1b + f2_ref[...]) + 0.5
    Eb_ref[...] = (A_ref[...] * e).astype(jnp.bfloat16)
    Eb = Eb_ref[...]
    EHs = jnp.dot(Eb, Hb_ref[...], preferred_element_type=jnp.float32)
    r = 1.0 / (EHs[:, dout:dout + 1] + 1e-8)
    C_ref[...] = Eb.astype(jnp.float32) * r
    out_ref[...] = EHs[:, :dout] * r


def _gat(Z, A, W, a_s, a_r):
    N = A.shape[0]
    din, dout = W.shape
    out, C = pl.pallas_call(
        _gat_body,
        grid=(N // BR,),
        in_specs=[
            pl.BlockSpec((N, din), lambda i: (0, 0)),
            pl.BlockSpec((BR, N), lambda i: (i, 0)),
            pl.BlockSpec((din, dout), lambda i: (0, 0)),
            pl.BlockSpec((1, dout), lambda i: (0, 0)),
            pl.BlockSpec((1, dout), lambda i: (0, 0)),
        ],
        out_specs=(
            pl.BlockSpec((BR, dout), lambda i: (i, 0)),
            pl.BlockSpec((BR, N), lambda i: (i, 0)),
        ),
        out_shape=(
            jax.ShapeDtypeStruct((N, dout), jnp.float32),
            jax.ShapeDtypeStruct((N, N), jnp.float32),
        ),
        scratch_shapes=[
            pltpu.VMEM((N, dout + 128), jnp.bfloat16),
            pltpu.VMEM((N, 1), jnp.float32),
            pltpu.VMEM((1, N), jnp.float32),
            pltpu.VMEM((BR, N), jnp.bfloat16),
        ],
    )(Z, A, W, a_s.reshape(1, -1), a_r.reshape(1, -1))
    return out, C


def _gat_an_body(Z_ref, A_ref, W_ref, as_ref, ar_ref, P_ref,
                 out_ref, C_ref, An_ref, Hb_ref, f1_ref, f2_ref,
                 Eb_ref):
    i = pl.program_id(0)
    dout = out_ref.shape[1]

    @pl.when(i == 0)
    def _prologue():
        H = jnp.dot(Z_ref[...], W_ref[...], preferred_element_type=jnp.float32)
        n = H.shape[0]
        ones_col = (lax.broadcasted_iota(jnp.int32, (n, 128), 1) == 0)
        Hb_ref[...] = jnp.concatenate(
            [H.astype(jnp.bfloat16), ones_col.astype(jnp.bfloat16)], axis=1)
        f1_ref[...] = 0.5 * lax.dot_general(
            H, as_ref[...], (((1,), (1,)), ((), ())),
            preferred_element_type=jnp.float32)
        f2_ref[...] = 0.5 * lax.dot_general(
            ar_ref[...], H, (((1,), (1,)), ((), ())),
            preferred_element_type=jnp.float32)

    f1b = f1_ref[pl.ds(i * BR, BR), :]
    e = 0.5 * jnp.tanh(f1b + f2_ref[...]) + 0.5
    Eb_ref[...] = (A_ref[...] * e).astype(jnp.bfloat16)
    Eb = Eb_ref[...]
    EHs = jnp.dot(Eb, Hb_ref[...], preferred_element_type=jnp.float32)
    r = 1.0 / (EHs[:, dout:dout + 1] + 1e-8)
    C_ref[...] = Eb.astype(jnp.float32) * r
    out_ref[...] = EHs[:, :dout] * r

    AP = jnp.dot(A_ref[...], P_ref[...].astype(jnp.bfloat16),
                 preferred_element_type=jnp.float32)
    Pb = P_ref[pl.ds(i * BR, BR), :]
    contrib = lax.dot_general(Pb, AP, (((0,), (0,)), ((), ())),
                              preferred_element_type=jnp.float32)

    @pl.when(i == 0)
    def _init():
        An_ref[...] = contrib

    @pl.when(i > 0)
    def _acc():
        An_ref[...] += contrib


def _gat_with_an(Z, A, W, a_s, a_r, P):
    N = A.shape[0]
    din, dout = W.shape
    c = P.shape[1]
    out, C, An = pl.pallas_call(
        _gat_an_body,
        grid=(N // BR,),
        in_specs=[
            pl.BlockSpec((N, din), lambda i: (0, 0)),
            pl.BlockSpec((BR, N), lambda i: (i, 0)),
            pl.BlockSpec((din, dout), lambda i: (0, 0)),
            pl.BlockSpec((1, dout), lambda i: (0, 0)),
            pl.BlockSpec((1, dout), lambda i: (0, 0)),
            pl.BlockSpec((N, c), lambda i: (0, 0)),
        ],
        out_specs=(
            pl.BlockSpec((BR, dout), lambda i: (i, 0)),
            pl.BlockSpec((BR, N), lambda i: (i, 0)),
            pl.BlockSpec((c, c), lambda i: (0, 0)),
        ),
        out_shape=(
            jax.ShapeDtypeStruct((N, dout), jnp.float32),
            jax.ShapeDtypeStruct((N, N), jnp.float32),
            jax.ShapeDtypeStruct((c, c), jnp.float32),
        ),
        scratch_shapes=[
            pltpu.VMEM((N, dout + 128), jnp.bfloat16),
            pltpu.VMEM((N, 1), jnp.float32),
            pltpu.VMEM((1, N), jnp.float32),
            pltpu.VMEM((BR, N), jnp.bfloat16),
        ],
    )(Z, A, W, a_s.reshape(1, -1), a_r.reshape(1, -1), P)
    return out, C, An


BRL = 256


def _gat1_body(Z_ref, A_ref, W_ref, as_ref, ar_ref, out_ref, C_ref, Ab_ref,
               Hb_ref, f1_ref, f2_ref, Eb_ref):
    i = pl.program_id(0)
    dout = out_ref.shape[1]

    @pl.when(i == 0)
    def _prologue():
        H = jnp.dot(Z_ref[...], W_ref[...], preferred_element_type=jnp.float32)
        n = H.shape[0]
        ones_col = (lax.broadcasted_iota(jnp.int32, (n, 128), 1) == 0)
        Hb_ref[...] = jnp.concatenate(
            [H.astype(jnp.bfloat16), ones_col.astype(jnp.bfloat16)], axis=1)
        f1_ref[...] = 0.5 * lax.dot_general(
            H, as_ref[...], (((1,), (1,)), ((), ())),
            preferred_element_type=jnp.float32)
        f2_ref[...] = 0.5 * lax.dot_general(
            ar_ref[...], H, (((1,), (1,)), ((), ())),
            preferred_element_type=jnp.float32)

    A_blk = A_ref[...]
    Ab_ref[...] = A_blk.astype(jnp.bfloat16)
    f1b = f1_ref[pl.ds(i * BRL, BRL), :]
    e = 0.5 * jnp.tanh(f1b + f2_ref[...]) + 0.5
    Eb_ref[...] = (A_blk * e).astype(jnp.bfloat16)
    Eb = Eb_ref[...]
    EHs = jnp.dot(Eb, Hb_ref[...], preferred_element_type=jnp.float32)
    r = 1.0 / (EHs[:, dout:dout + 1] + 1e-8)
    C_ref[...] = Eb.astype(jnp.float32) * r
    out_ref[...] = EHs[:, :dout] * r


def _gat1(Z, A, W, a_s, a_r):
    N = A.shape[0]
    din, dout = W.shape
    out, C, Ab = pl.pallas_call(
        _gat1_body,
        grid=(N // BRL,),
        in_specs=[
            pl.BlockSpec((N, din), lambda i: (0, 0)),
            pl.BlockSpec((BRL, N), lambda i: (i, 0)),
            pl.BlockSpec((din, dout), lambda i: (0, 0)),
            pl.BlockSpec((1, dout), lambda i: (0, 0)),
            pl.BlockSpec((1, dout), lambda i: (0, 0)),
        ],
        out_specs=(
            pl.BlockSpec((BRL, dout), lambda i: (i, 0)),
            pl.BlockSpec((BRL, N), lambda i: (i, 0)),
            pl.BlockSpec((BRL, N), lambda i: (i, 0)),
        ),
        out_shape=(
            jax.ShapeDtypeStruct((N, dout), jnp.float32),
            jax.ShapeDtypeStruct((N, N), jnp.float32),
            jax.ShapeDtypeStruct((N, N), jnp.bfloat16),
        ),
        scratch_shapes=[
            pltpu.VMEM((N, dout + 128), jnp.bfloat16),
            pltpu.VMEM((N, 1), jnp.float32),
            pltpu.VMEM((1, N), jnp.float32),
            pltpu.VMEM((BRL, N), jnp.bfloat16),
        ],
    )(Z, A, W, a_s.reshape(1, -1), a_r.reshape(1, -1))
    return out, C, Ab


def _ahat_body(Z_ref, g_ref, b_ref, out_ref, zs_ref, M_ref):
    i = pl.program_id(0)
    n = Z_ref.shape[0]

    @pl.when(i == 0)
    def _prologue():
        Z = Z_ref[...]
        zs_ref[...] = jnp.sum(Z, axis=0, keepdims=True)
        M_ref[...] = lax.dot_general(Z, Z, (((0,), (0,)), ((), ())),
                                     preferred_element_type=jnp.float32)

    Zb = Z_ref[pl.ds(i * BR, BR), :]
    G = lax.dot_general(Zb, Z_ref[...], (((1,), (1,)), ((), ())),
                        preferred_element_type=jnp.float32)
    inv_n = 1.0 / n
    mu = lax.dot_general(Zb, zs_ref[...], (((1,), (1,)), ((), ())),
                         preferred_element_type=jnp.float32) * inv_n
    Q = jnp.dot(Zb, M_ref[...], preferred_element_type=jnp.float32)
    m2 = jnp.sum(Q * Zb, axis=1, keepdims=True) * inv_n
    var = m2 - mu * mu
    k = lax.rsqrt(var + 1e-5) * 0.5
    y = (G - mu) * k * g_ref[...] + 0.5 * b_ref[...]
    out_ref[...] = 0.5 * jnp.tanh(y) + 0.5


def _ahat(Z, g, b):
    N = Z.shape[0]
    h = Z.shape[1]
    return pl.pallas_call(
        _ahat_body,
        grid=(N // BR,),
        in_specs=[
            pl.BlockSpec((N, h), lambda i: (0, 0)),
            pl.BlockSpec((1, N), lambda i: (0, 0)),
            pl.BlockSpec((1, N), lambda i: (0, 0)),
        ],
        out_specs=pl.BlockSpec((BR, N), lambda i: (i, 0)),
        out_shape=jax.ShapeDtypeStruct((N, N), jnp.float32),
        scratch_shapes=[
            pltpu.VMEM((1, h), jnp.float32),
            pltpu.VMEM((h, h), jnp.float32),
        ],
    )(Z, g.reshape(1, -1), b.reshape(1, -1))


def kernel(X, A, params):
    Z, C1, Ab = _gat1(X, A, params['We0'], params['ase0'], params['are0'])
    enc_attn = [C1]
    for li in range(1, 3):
        Z, C = _gat(Z, Ab, params['We%d' % li], params['ase%d' % li],
                    params['are%d' % li])
        enc_attn.append(C)

    A_hat = _ahat(Z, params['g_ln'], params['b_ln'])

    P0 = jax.nn.softmax(Z @ params['Wc0'] + params['bc0'], axis=1)
    S0 = jnp.argmax(P0, axis=1)

    dec_attn = []
    Xd, C, An1 = _gat_with_an(Z, Ab, params['Wd0'], params['asd0'],
                              params['ard0'], P0)
    dec_attn.append(C)
    for li in range(1, 3):
        Xd, C = _gat(Xd, Ab, params['Wd%d' % li], params['asd%d' % li],
                     params['ard%d' % li])
        dec_attn.append(C)
    X_hat = Xd

    Xn1 = P0.T @ Z

    P1 = jax.nn.softmax(Xn1 @ params['Wc1'] + params['bc1'], axis=1)
    S1 = jnp.argmax(P1, axis=1)
    Xn2 = P1.T @ Xn1
    An2 = P1.T @ An1 @ P1

    X_all_final = [Z, Xn1, Xn2]
    A_all_final = [A, An1, An2]
    P_all = [P0, P1]
    S_all = [S0, S1]
    return (X_hat, A_hat, X_all_final, A_all_final, P_all, S_all,
            [enc_attn, dec_attn])


# speedup vs baseline: 1.0318x; 1.0318x over previous
import jax
import jax.numpy as jnp
from jax import lax
from jax.experimental import pallas as pl
from jax.experimental.pallas import tpu as pltpu

BR = 512


def _gat_body(Z_ref, A_ref, W_ref, as_ref, ar_ref, out_ref, C_ref,
              Hb_ref, f1_ref, f2_ref, Eb_ref):
    i = pl.program_id(0)
    dout = out_ref.shape[1]

    @pl.when(i == 0)
    def _prologue():
        H = jnp.dot(Z_ref[...], W_ref[...], preferred_element_type=jnp.float32)
        n = H.shape[0]
        ones_col = (lax.broadcasted_iota(jnp.int32, (n, 128), 1) == 0)
        Hb_ref[...] = jnp.concatenate(
            [H.astype(jnp.bfloat16), ones_col.astype(jnp.bfloat16)], axis=1)
        f1_ref[...] = 0.5 * lax.dot_general(
            H, as_ref[...], (((1,), (1,)), ((), ())),
            preferred_element_type=jnp.float32)
        f2_ref[...] = 0.5 * lax.dot_general(
            ar_ref[...], H, (((1,), (1,)), ((), ())),
            preferred_element_type=jnp.float32)

    f1b = f1_ref[pl.ds(i * BR, BR), :]
    e = 0.5 * jnp.tanh(f1b + f2_ref[...]) + 0.5
    Eb_ref[...] = (A_ref[...] * e).astype(jnp.bfloat16)
    Eb = Eb_ref[...]
    EHs = jnp.dot(Eb, Hb_ref[...], preferred_element_type=jnp.float32)
    r = 1.0 / (EHs[:, dout:dout + 1] + 1e-8)
    C_ref[...] = Eb.astype(jnp.float32) * r
    out_ref[...] = EHs[:, :dout] * r


def _gat(Z, A, W, a_s, a_r):
    N = A.shape[0]
    din, dout = W.shape
    out, C = pl.pallas_call(
        _gat_body,
        grid=(N // BR,),
        in_specs=[
            pl.BlockSpec((N, din), lambda i: (0, 0)),
            pl.BlockSpec((BR, N), lambda i: (i, 0)),
            pl.BlockSpec((din, dout), lambda i: (0, 0)),
            pl.BlockSpec((1, dout), lambda i: (0, 0)),
            pl.BlockSpec((1, dout), lambda i: (0, 0)),
        ],
        out_specs=(
            pl.BlockSpec((BR, dout), lambda i: (i, 0)),
            pl.BlockSpec((BR, N), lambda i: (i, 0)),
        ),
        out_shape=(
            jax.ShapeDtypeStruct((N, dout), jnp.float32),
            jax.ShapeDtypeStruct((N, N), jnp.float32),
        ),
        scratch_shapes=[
            pltpu.VMEM((N, dout + 128), jnp.bfloat16),
            pltpu.VMEM((N, 1), jnp.float32),
            pltpu.VMEM((1, N), jnp.float32),
            pltpu.VMEM((BR, N), jnp.bfloat16),
        ],
    )(Z, A, W, a_s.reshape(1, -1), a_r.reshape(1, -1))
    return out, C


def _gat_an_body(Z_ref, A_ref, W_ref, as_ref, ar_ref, P_ref,
                 out_ref, C_ref, An_ref, Hb_ref, f1_ref, f2_ref,
                 Eb_ref):
    i = pl.program_id(0)
    dout = out_ref.shape[1]

    @pl.when(i == 0)
    def _prologue():
        H = jnp.dot(Z_ref[...], W_ref[...], preferred_element_type=jnp.float32)
        n = H.shape[0]
        ones_col = (lax.broadcasted_iota(jnp.int32, (n, 128), 1) == 0)
        Hb_ref[...] = jnp.concatenate(
            [H.astype(jnp.bfloat16), ones_col.astype(jnp.bfloat16)], axis=1)
        f1_ref[...] = 0.5 * lax.dot_general(
            H, as_ref[...], (((1,), (1,)), ((), ())),
            preferred_element_type=jnp.float32)
        f2_ref[...] = 0.5 * lax.dot_general(
            ar_ref[...], H, (((1,), (1,)), ((), ())),
            preferred_element_type=jnp.float32)

    f1b = f1_ref[pl.ds(i * BR, BR), :]
    e = 0.5 * jnp.tanh(f1b + f2_ref[...]) + 0.5
    Eb_ref[...] = (A_ref[...] * e).astype(jnp.bfloat16)
    Eb = Eb_ref[...]
    EHs = jnp.dot(Eb, Hb_ref[...], preferred_element_type=jnp.float32)
    r = 1.0 / (EHs[:, dout:dout + 1] + 1e-8)
    C_ref[...] = Eb.astype(jnp.float32) * r
    out_ref[...] = EHs[:, :dout] * r

    AP = jnp.dot(A_ref[...], P_ref[...].astype(jnp.bfloat16),
                 preferred_element_type=jnp.float32)
    Pb = P_ref[pl.ds(i * BR, BR), :]
    contrib = lax.dot_general(Pb, AP, (((0,), (0,)), ((), ())),
                              preferred_element_type=jnp.float32)

    @pl.when(i == 0)
    def _init():
        An_ref[...] = contrib

    @pl.when(i > 0)
    def _acc():
        An_ref[...] += contrib


def _gat_with_an(Z, A, W, a_s, a_r, P):
    N = A.shape[0]
    din, dout = W.shape
    c = P.shape[1]
    out, C, An = pl.pallas_call(
        _gat_an_body,
        grid=(N // BR,),
        in_specs=[
            pl.BlockSpec((N, din), lambda i: (0, 0)),
            pl.BlockSpec((BR, N), lambda i: (i, 0)),
            pl.BlockSpec((din, dout), lambda i: (0, 0)),
            pl.BlockSpec((1, dout), lambda i: (0, 0)),
            pl.BlockSpec((1, dout), lambda i: (0, 0)),
            pl.BlockSpec((N, c), lambda i: (0, 0)),
        ],
        out_specs=(
            pl.BlockSpec((BR, dout), lambda i: (i, 0)),
            pl.BlockSpec((BR, N), lambda i: (i, 0)),
            pl.BlockSpec((c, c), lambda i: (0, 0)),
        ),
        out_shape=(
            jax.ShapeDtypeStruct((N, dout), jnp.float32),
            jax.ShapeDtypeStruct((N, N), jnp.float32),
            jax.ShapeDtypeStruct((c, c), jnp.float32),
        ),
        scratch_shapes=[
            pltpu.VMEM((N, dout + 128), jnp.bfloat16),
            pltpu.VMEM((N, 1), jnp.float32),
            pltpu.VMEM((1, N), jnp.float32),
            pltpu.VMEM((BR, N), jnp.bfloat16),
        ],
    )(Z, A, W, a_s.reshape(1, -1), a_r.reshape(1, -1), P)
    return out, C, An


BRL = 256


def _gat1_body(Z_ref, A_ref, W_ref, as_ref, ar_ref, out_ref, C_ref, Ab_ref,
               Hb_ref, f1_ref, f2_ref, Eb_ref):
    i = pl.program_id(0)
    dout = out_ref.shape[1]

    @pl.when(i == 0)
    def _prologue():
        H = jnp.dot(Z_ref[...], W_ref[...], preferred_element_type=jnp.float32)
        n = H.shape[0]
        ones_col = (lax.broadcasted_iota(jnp.int32, (n, 128), 1) == 0)
        Hb_ref[...] = jnp.concatenate(
            [H.astype(jnp.bfloat16), ones_col.astype(jnp.bfloat16)], axis=1)
        f1_ref[...] = 0.5 * lax.dot_general(
            H, as_ref[...], (((1,), (1,)), ((), ())),
            preferred_element_type=jnp.float32)
        f2_ref[...] = 0.5 * lax.dot_general(
            ar_ref[...], H, (((1,), (1,)), ((), ())),
            preferred_element_type=jnp.float32)

    A_blk = A_ref[...]
    Ab_ref[...] = A_blk.astype(jnp.bfloat16)
    f1b = f1_ref[pl.ds(i * BRL, BRL), :]
    e = 0.5 * jnp.tanh(f1b + f2_ref[...]) + 0.5
    Eb_ref[...] = (A_blk * e).astype(jnp.bfloat16)
    Eb = Eb_ref[...]
    EHs = jnp.dot(Eb, Hb_ref[...], preferred_element_type=jnp.float32)
    r = 1.0 / (EHs[:, dout:dout + 1] + 1e-8)
    C_ref[...] = Eb.astype(jnp.float32) * r
    out_ref[...] = EHs[:, :dout] * r


def _gat1(Z, A, W, a_s, a_r):
    N = A.shape[0]
    din, dout = W.shape
    out, C, Ab = pl.pallas_call(
        _gat1_body,
        grid=(N // BRL,),
        in_specs=[
            pl.BlockSpec((N, din), lambda i: (0, 0)),
            pl.BlockSpec((BRL, N), lambda i: (i, 0)),
            pl.BlockSpec((din, dout), lambda i: (0, 0)),
            pl.BlockSpec((1, dout), lambda i: (0, 0)),
            pl.BlockSpec((1, dout), lambda i: (0, 0)),
        ],
        out_specs=(
            pl.BlockSpec((BRL, dout), lambda i: (i, 0)),
            pl.BlockSpec((BRL, N), lambda i: (i, 0)),
            pl.BlockSpec((BRL, N), lambda i: (i, 0)),
        ),
        out_shape=(
            jax.ShapeDtypeStruct((N, dout), jnp.float32),
            jax.ShapeDtypeStruct((N, N), jnp.float32),
            jax.ShapeDtypeStruct((N, N), jnp.bfloat16),
        ),
        scratch_shapes=[
            pltpu.VMEM((N, dout + 128), jnp.bfloat16),
            pltpu.VMEM((N, 1), jnp.float32),
            pltpu.VMEM((1, N), jnp.float32),
            pltpu.VMEM((BRL, N), jnp.bfloat16),
        ],
    )(Z, A, W, a_s.reshape(1, -1), a_r.reshape(1, -1))
    return out, C, Ab


def _ahat_body(Z_ref, g_ref, b_ref, out_ref):
    i = pl.program_id(0)
    Zb = Z_ref[pl.ds(i * BR, BR), :]
    G = lax.dot_general(Zb, Z_ref[...], (((1,), (1,)), ((), ())),
                        preferred_element_type=jnp.float32)
    mu = jnp.mean(G, axis=1, keepdims=True)
    m2 = jnp.mean(G * G, axis=1, keepdims=True)
    var = m2 - mu * mu
    k = lax.rsqrt(var + 1e-5) * 0.5
    y = (G - mu) * k * g_ref[...] + 0.5 * b_ref[...]
    out_ref[...] = 0.5 * jnp.tanh(y) + 0.5


def _ahat(Z, g, b):
    N = Z.shape[0]
    h = Z.shape[1]
    return pl.pallas_call(
        _ahat_body,
        grid=(N // BR,),
        in_specs=[
            pl.BlockSpec((N, h), lambda i: (0, 0)),
            pl.BlockSpec((1, N), lambda i: (0, 0)),
            pl.BlockSpec((1, N), lambda i: (0, 0)),
        ],
        out_specs=pl.BlockSpec((BR, N), lambda i: (i, 0)),
        out_shape=jax.ShapeDtypeStruct((N, N), jnp.float32),
    )(Z, g.reshape(1, -1), b.reshape(1, -1))


def kernel(X, A, params):
    Z, C1, Ab = _gat1(X, A, params['We0'], params['ase0'], params['are0'])
    enc_attn = [C1]
    for li in range(1, 3):
        Z, C = _gat(Z, Ab, params['We%d' % li], params['ase%d' % li],
                    params['are%d' % li])
        enc_attn.append(C)

    A_hat = _ahat(Z, params['g_ln'], params['b_ln'])

    P0 = jax.nn.softmax(Z @ params['Wc0'] + params['bc0'], axis=1)
    S0 = jnp.argmax(P0, axis=1)

    dec_attn = []
    Xd, C, An1 = _gat_with_an(Z, Ab, params['Wd0'], params['asd0'],
                              params['ard0'], P0)
    dec_attn.append(C)
    for li in range(1, 3):
        Xd, C = _gat(Xd, Ab, params['Wd%d' % li], params['asd%d' % li],
                     params['ard%d' % li])
        dec_attn.append(C)
    X_hat = Xd

    Xn1 = P0.T @ Z

    P1 = jax.nn.softmax(Xn1 @ params['Wc1'] + params['bc1'], axis=1)
    S1 = jnp.argmax(P1, axis=1)
    Xn2 = P1.T @ Xn1
    An2 = P1.T @ An1 @ P1

    X_all_final = [Z, Xn1, Xn2]
    A_all_final = [A, An1, An2]
    P_all = [P0, P1]
    S_all = [S0, S1]
    return (X_hat, A_hat, X_all_final, A_all_final, P_all, S_all,
            [enc_attn, dec_attn])
